# baseline (speedup 1.0000x reference)
import functools

import jax
import jax.numpy as jnp
from jax import lax
from jax.experimental import pallas as pl
from jax.experimental.pallas import tpu as pltpu
from jax.experimental.pallas import tpu_sc as plsc

_LANES = 16
_IDX_COLS = 128


@functools.lru_cache(maxsize=None)
def _make_embed(B, L, V, D, interpret=False):
    N = B * L
    NC, NS = 2, 16
    NW = NC * NS
    assert N % (NW * _IDX_COLS) == 0
    rows_per_w = N // NW
    chunk = 2 * _IDX_COLS
    n_chunk = rows_per_w // chunk
    idxr_per_w = rows_per_w // _IDX_COLS
    assert D % _LANES == 0
    KD = D // _LANES
    NBUF = 2

    mesh = plsc.VectorSubcoreMesh(
        core_axis_name="c", subcore_axis_name="s", num_cores=NC, num_subcores=NS)

    def body(ids_hbm, tok_hbm, pos_hbm, g_hbm, bt_hbm, out_hbm,
             idx_v, pos_v, g_v, bt_v, buf, sem_g, sem_o):
        cid = lax.axis_index("c")
        sid = lax.axis_index("s")
        wid = sid * NC + cid
        base_row = wid * rows_per_w

        pltpu.sync_copy(ids_hbm.at[wid], idx_v)
        pltpu.sync_copy(pos_hbm.at[pl.ds(0, L)], pos_v)
        pltpu.sync_copy(g_hbm, g_v)
        pltpu.sync_copy(bt_hbm, bt_v)

        gvs = [g_v[pl.ds(_LANES * k, _LANES)] for k in range(KD)]
        bvs = [bt_v[pl.ds(_LANES * k, _LANES)] for k in range(KD)]

        def gather_descs(c, slot):
            return [
                pltpu.make_async_copy(
                    tok_hbm.at[idx_v.at[2 * c + h]],
                    buf.at[slot, pl.ds(h * _IDX_COLS, _IDX_COLS)], sem_g)
                for h in range(2)
            ]

        def out_desc(c, slot, h):
            off = pl.multiple_of(base_row + c * chunk + h * _IDX_COLS, _IDX_COLS)
            return pltpu.make_async_copy(
                buf.at[slot, pl.ds(h * _IDX_COLS, _IDX_COLS)],
                out_hbm.at[pl.ds(off, _IDX_COLS)], sem_o)

        for d in gather_descs(0, 0):
            d.start()

        @pl.loop(0, n_chunk)
        def _chunk(c):
            cur = lax.rem(c, NBUF)
            nxt = 1 - cur
            for d in gather_descs(c, cur):
                d.wait()

            @pl.when(c + 1 < n_chunk)
            def _prefetch():
                @pl.when(c >= 1)
                def _free_buf():
                    out_desc(c - 1, nxt, 0).wait()
                    out_desc(c - 1, nxt, 1).wait()

                for d in gather_descs(c + 1, nxt):
                    d.start()

            lbase = lax.rem(c * chunk, L)

            def _half_rows(h):
                @plsc.parallel_loop(h * _IDX_COLS, (h + 1) * _IDX_COLS, unroll=8)
                def _row(r):
                    _row_body(r)
                out_desc(c, cur, h).start()

            def _row_body(r):
                lpos = lax.rem(lbase + r, L)
                t = [buf[cur, r, pl.ds(_LANES * k, _LANES)]
                     + pos_v[lpos, pl.ds(_LANES * k, _LANES)]
                     for k in range(KD)]
                sv = ((t[0] + t[1]) + (t[2] + t[3])) + ((t[4] + t[5]) + (t[6] + t[7]))
                qv = (((t[0] * t[0] + t[1] * t[1]) + (t[2] * t[2] + t[3] * t[3]))
                      + ((t[4] * t[4] + t[5] * t[5]) + (t[6] * t[6] + t[7] * t[7])))
                lanes = lax.iota(jnp.int32, _LANES)
                for sh in (8, 4, 2, 1):
                    perm = lanes ^ sh
                    sv = sv + sv.at[perm].get(mode="promise_in_bounds", unique_indices=True)
                    qv = qv + qv.at[perm].get(mode="promise_in_bounds", unique_indices=True)
                mean = sv * (1.0 / D)
                var = qv * (1.0 / D) - mean * mean + 1e-5
                i = lax.bitcast_convert_type(var, jnp.int32)
                i = jnp.int32(0x5F3759DF) - lax.shift_right_arithmetic(i, 1)
                y = lax.bitcast_convert_type(i, jnp.float32)
                y = y * (1.5 - 0.5 * var * y * y)
                y = y * (1.5 - 0.5 * var * y * y)
                y = y * (1.5 - 0.5 * var * y * y)
                a = y
                b = -mean * y
                for k in range(KD):
                    buf[cur, r, pl.ds(_LANES * k, _LANES)] = (t[k] * a + b) * gvs[k] + bvs[k]

            _half_rows(0)
            _half_rows(1)

        out_desc(n_chunk - 1, (n_chunk - 1) % NBUF, 0).wait()
        out_desc(n_chunk - 1, (n_chunk - 1) % NBUF, 1).wait()

    return pl.kernel(
        body,
        out_type=jax.ShapeDtypeStruct((N, D), jnp.float32),
        mesh=mesh,
        scratch_types=[
            pltpu.VMEM((idxr_per_w, _IDX_COLS), jnp.int32),
            pltpu.VMEM((L, D), jnp.float32),
            pltpu.VMEM((D,), jnp.float32),
            pltpu.VMEM((D,), jnp.float32),
            pltpu.VMEM((NBUF, chunk, D), jnp.float32),
            pltpu.SemaphoreType.DMA,
            pltpu.SemaphoreType.DMA,
        ],
        interpret=interpret,
    )


def kernel(input_ids, token_table, pos_table, ln_gamma, ln_beta):
    B, L = input_ids.shape
    V, D = token_table.shape
    NW = 32
    ids3d = input_ids.reshape(NW, B * L // (NW * _IDX_COLS), _IDX_COLS).astype(jnp.int32)
    fn = _make_embed(B, L, V, D)
    out = fn(ids3d, token_table, pos_table, ln_gamma, ln_beta)
    return out.reshape(B, L, D)


# speedup vs baseline: 1.2935x; 1.2935x over previous
import functools

import jax
import jax.numpy as jnp
from jax import lax
from jax.experimental import pallas as pl
from jax.experimental.pallas import tpu as pltpu
from jax.experimental.pallas import tpu_sc as plsc

_LANES = 16
_IDX_COLS = 128


@functools.lru_cache(maxsize=None)
def _make_embed(B, L, V, D, interpret=False):
    N = B * L
    NC, NS = 2, 16
    NW = NC * NS
    assert N % (NW * _IDX_COLS) == 0
    rows_per_w = N // NW
    chunk = _IDX_COLS
    n_chunk = rows_per_w // chunk
    idxr_per_w = rows_per_w // _IDX_COLS
    assert D % _LANES == 0
    KD = D // _LANES
    NBUF = 2

    mesh = plsc.VectorSubcoreMesh(
        core_axis_name="c", subcore_axis_name="s", num_cores=NC, num_subcores=NS)

    def body(ids_hbm, tok_hbm, pos_hbm, g_hbm, bt_hbm, out_hbm,
             idx_v, pos_v, g_v, bt_v, buf, sem_g, sem_o):
        cid = lax.axis_index("c")
        sid = lax.axis_index("s")
        wid = sid * NC + cid
        base_row = wid * rows_per_w

        pltpu.sync_copy(ids_hbm.at[wid], idx_v)
        pltpu.sync_copy(pos_hbm.at[pl.ds(0, L)], pos_v)
        pltpu.sync_copy(g_hbm, g_v)
        pltpu.sync_copy(bt_hbm, bt_v)

        gvs = [g_v[pl.ds(_LANES * k, _LANES)] for k in range(KD)]
        bvs = [bt_v[pl.ds(_LANES * k, _LANES)] for k in range(KD)]

        def gather_desc(c, slot):
            return pltpu.make_async_copy(
                tok_hbm.at[idx_v.at[c]], buf.at[slot], sem_g)

        def out_desc(c, slot):
            off = pl.multiple_of(base_row + c * chunk, chunk)
            return pltpu.make_async_copy(
                buf.at[slot], out_hbm.at[pl.ds(off, chunk)], sem_o)

        gather_desc(0, 0).start()

        @pl.loop(0, n_chunk)
        def _chunk(c):
            cur = lax.rem(c, NBUF)
            nxt = 1 - cur
            gather_desc(c, cur).wait()

            @pl.when(c + 1 < n_chunk)
            def _prefetch():
                @pl.when(c >= 1)
                def _free_buf():
                    out_desc(c - 1, nxt).wait()

                gather_desc(c + 1, nxt).start()

            lbase = lax.rem(c * chunk, L)

            @plsc.parallel_loop(0, chunk, unroll=16)
            def _row(r):
                lpos = lax.rem(lbase + r, L)
                t = [buf[cur, r, pl.ds(_LANES * k, _LANES)]
                     + pos_v[lpos, pl.ds(_LANES * k, _LANES)]
                     for k in range(KD)]
                sv = ((t[0] + t[1]) + (t[2] + t[3])) + ((t[4] + t[5]) + (t[6] + t[7]))
                qv = (((t[0] * t[0] + t[1] * t[1]) + (t[2] * t[2] + t[3] * t[3]))
                      + ((t[4] * t[4] + t[5] * t[5]) + (t[6] * t[6] + t[7] * t[7])))
                lanes = lax.iota(jnp.int32, _LANES)
                for sh in (8, 4, 2, 1):
                    perm = lanes ^ sh
                    sv = sv + sv.at[perm].get(mode="promise_in_bounds", unique_indices=True)
                    qv = qv + qv.at[perm].get(mode="promise_in_bounds", unique_indices=True)
                mean = sv * (1.0 / D)
                var = qv * (1.0 / D) - mean * mean + 1e-5
                i = lax.bitcast_convert_type(var, jnp.int32)
                i = jnp.int32(0x5F3759DF) - lax.shift_right_arithmetic(i, 1)
                y = lax.bitcast_convert_type(i, jnp.float32)
                y = y * (1.5 - 0.5 * var * y * y)
                y = y * (1.5 - 0.5 * var * y * y)
                y = y * (1.5 - 0.5 * var * y * y)
                a = y
                b = -mean * y
                for k in range(KD):
                    buf[cur, r, pl.ds(_LANES * k, _LANES)] = (t[k] * a + b) * gvs[k] + bvs[k]

            out_desc(c, cur).start()

        out_desc(n_chunk - 1, (n_chunk - 1) % NBUF).wait()

    return pl.kernel(
        body,
        out_type=jax.ShapeDtypeStruct((N, D), jnp.float32),
        mesh=mesh,
        scratch_types=[
            pltpu.VMEM((idxr_per_w, _IDX_COLS), jnp.int32),
            pltpu.VMEM((L, D), jnp.float32),
            pltpu.VMEM((D,), jnp.float32),
            pltpu.VMEM((D,), jnp.float32),
            pltpu.VMEM((NBUF, chunk, D), jnp.float32),
            pltpu.SemaphoreType.DMA,
            pltpu.SemaphoreType.DMA,
        ],
        interpret=interpret,
    )


def kernel(input_ids, token_table, pos_table, ln_gamma, ln_beta):
    B, L = input_ids.shape
    V, D = token_table.shape
    NW = 32
    ids3d = input_ids.reshape(NW, B * L // (NW * _IDX_COLS), _IDX_COLS).astype(jnp.int32)
    fn = _make_embed(B, L, V, D)
    out = fn(ids3d, token_table, pos_table, ln_gamma, ln_beta)
    return out.reshape(B, L, D)
